# Initial kernel scaffold; baseline (speedup 1.0000x reference)
import functools

import jax
import jax.numpy as jnp
from jax import lax
from jax.experimental import pallas as pl
from jax.experimental.pallas import tpu as pltpu
from jax.experimental.pallas import tpu_sc as plsc

_N = 10000
_E = 320000
_NC = 2
_NS = 16
_NW = _NC * _NS
_EPW = _E // _NW
_C = 80
_NCH = _EPW // _C
_RPT = 640
_NPAD = _RPT * _NS


def _seg_sum(feat, src3, dst3, d, with_cnt):
  mesh = plsc.VectorSubcoreMesh(core_axis_name="c", subcore_axis_name="s")
  outs = [jax.ShapeDtypeStruct((_NC, _NPAD, d), jnp.float32)]
  scratch = [
      pltpu.VMEM((_NCH, _C), jnp.int32),
      pltpu.VMEM((_NCH, _C), jnp.int32),
      pltpu.VMEM((_C, d), jnp.float32),
      pltpu.VMEM_SHARED((_NPAD, d), jnp.float32),
      pltpu.SemaphoreType.DMA,
  ]
  inputs = [feat, src3, dst3, jnp.zeros((_NPAD, d), jnp.float32)]
  if with_cnt:
    outs.append(jax.ShapeDtypeStruct((_NC, _NPAD, 8), jnp.float32))
    scratch += [
        pltpu.VMEM((_C, 8), jnp.float32),
        pltpu.VMEM_SHARED((_NPAD, 8), jnp.float32),
    ]
    inputs += [jnp.ones((_C, 8), jnp.float32),
               jnp.zeros((_NPAD, 8), jnp.float32)]

  def body(*refs):
    if with_cnt:
      (feat_h, src_h, dst_h, zf_h, ones_h, zc_h, agg_o, cnt_o,
       sidx, didx, rows, acc, sem, ones_v, cacc) = refs
    else:
      (feat_h, src_h, dst_h, zf_h, agg_o,
       sidx, didx, rows, acc, sem) = refs
    c = lax.axis_index("c")
    s = lax.axis_index("s")
    wid = c * _NS + s
    r0 = s * _RPT

    pltpu.sync_copy(zf_h.at[pl.ds(r0, _RPT)], acc.at[pl.ds(r0, _RPT)])
    if with_cnt:
      pltpu.sync_copy(zc_h.at[pl.ds(r0, _RPT)], cacc.at[pl.ds(r0, _RPT)])
      pltpu.sync_copy(ones_h, ones_v)
    pltpu.sync_copy(src_h.at[wid], sidx)
    pltpu.sync_copy(dst_h.at[wid], didx)
    plsc.subcore_barrier()

    def chunk(j, carry):
      pltpu.async_copy(feat_h.at[sidx.at[j]], rows, sem).wait()
      pltpu.sync_copy(rows, acc.at[didx.at[j]], add=True)
      if with_cnt:
        pltpu.sync_copy(ones_v, cacc.at[didx.at[j]], add=True)
      return carry

    lax.fori_loop(0, _NCH, chunk, 0)
    plsc.subcore_barrier()

    pltpu.sync_copy(acc.at[pl.ds(r0, _RPT)], agg_o.at[c, pl.ds(r0, _RPT)])
    if with_cnt:
      pltpu.sync_copy(cacc.at[pl.ds(r0, _RPT)], cnt_o.at[c, pl.ds(r0, _RPT)])

  f = pl.kernel(body, out_type=tuple(outs), mesh=mesh,
                scratch_types=tuple(scratch))
  return f(*inputs)


def _dot_t(a, w):
  return lax.dot_general(a, w, (((1,), (1,)), ((), ())),
                         precision=lax.Precision.HIGHEST,
                         preferred_element_type=jnp.float32)


def _mean(agg_r, cnt_r):
  cnt = cnt_r[0, :, 0:1] + cnt_r[1, :, 0:1]
  inv = 1.0 / jnp.maximum(cnt, 1.0)
  return (agg_r[0] + agg_r[1]) * inv


def _elu(h):
  return jnp.where(h > 0, h, jnp.expm1(jnp.minimum(h, 0.0)))


_BM = 1000


def _t1_body(x_r, a_r, c_r, w1l_r, b1_r, w1r_r, w2l_r, b2_r, w2r_r,
             p2_o, r2_o):
  mean = _mean(a_r, c_r)
  h1 = _dot_t(mean, w1l_r[...]) + b1_r[...] + _dot_t(x_r[...], w1r_r[...])
  h1 = _elu(h1)
  p2_o[...] = _dot_t(h1, w2l_r[...])
  r2_o[...] = _dot_t(h1, w2r_r[...]) + b2_r[...]


def _t1(x, agg1, cnt, w1l, b1, w1r, w2l, b2, w2r):
  grid = (_N // _BM,)
  full = lambda shape: pl.BlockSpec(shape, lambda i: (0,) * len(shape))
  return pl.pallas_call(
      _t1_body,
      grid=grid,
      in_specs=[
          pl.BlockSpec((_BM, 128), lambda i: (i, 0)),
          pl.BlockSpec((_NC, _BM, 128), lambda i: (0, i, 0)),
          pl.BlockSpec((_NC, _BM, 8), lambda i: (0, i, 0)),
          full((256, 128)), full((1, 256)), full((256, 128)),
          full((32, 256)), full((1, 32)), full((32, 256)),
      ],
      out_specs=[
          pl.BlockSpec((_BM, 32), lambda i: (i, 0)),
          pl.BlockSpec((_BM, 32), lambda i: (i, 0)),
      ],
      out_shape=[
          jax.ShapeDtypeStruct((_N, 32), jnp.float32),
          jax.ShapeDtypeStruct((_N, 32), jnp.float32),
      ],
  )(x, agg1, cnt, w1l, b1, w1r, w2l, b2, w2r)


def _t2_body(a_r, c_r, r2_r, h2_o):
  h2_o[...] = _elu(_mean(a_r, c_r) + r2_r[...])


def _t2(agg2, cnt, r2):
  grid = (_N // _BM,)
  return pl.pallas_call(
      _t2_body,
      grid=grid,
      in_specs=[
          pl.BlockSpec((_NC, _BM, 32), lambda i: (0, i, 0)),
          pl.BlockSpec((_NC, _BM, 8), lambda i: (0, i, 0)),
          pl.BlockSpec((_BM, 32), lambda i: (i, 0)),
      ],
      out_specs=pl.BlockSpec((_BM, 32), lambda i: (i, 0)),
      out_shape=jax.ShapeDtypeStruct((_N, 32), jnp.float32),
  )(agg2, cnt, r2)


def _t3_body(a_r, c_r, h2_r, w3l_r, b3_r, w3r_r, out_o):
  mean = _mean(a_r, c_r)
  out_o[...] = (_dot_t(mean, w3l_r[...]) + b3_r[...]
                + _dot_t(h2_r[...], w3r_r[...]))


def _t3(agg3, cnt, h2, w3l, b3, w3r):
  grid = (_N // _BM,)
  full = lambda shape: pl.BlockSpec(shape, lambda i: (0,) * len(shape))
  return pl.pallas_call(
      _t3_body,
      grid=grid,
      in_specs=[
          pl.BlockSpec((_NC, _BM, 32), lambda i: (0, i, 0)),
          pl.BlockSpec((_NC, _BM, 8), lambda i: (0, i, 0)),
          pl.BlockSpec((_BM, 32), lambda i: (i, 0)),
          full((64, 32)), full((1, 64)), full((64, 32)),
      ],
      out_specs=pl.BlockSpec((_BM, 64), lambda i: (i, 0)),
      out_shape=jax.ShapeDtypeStruct((_N, 64), jnp.float32),
  )(agg3, cnt, h2, w3l, b3, w3r)


def _pad_rows(a):
  return jnp.pad(a, ((0, _NPAD - _N), (0, 0)))


def kernel(x, edge_index, W1l, b1, W1r, W2l, b2, W2r, W3l, b3, W3r):
  src3 = edge_index[0].astype(jnp.int32).reshape(_NW, _NCH, _C)
  dst3 = edge_index[1].astype(jnp.int32).reshape(_NW, _NCH, _C)

  agg1, cnt = _seg_sum(_pad_rows(x), src3, dst3, 128, True)
  p2, r2 = _t1(x, agg1, cnt, W1l, b1.reshape(1, 256), W1r,
               W2l, b2.reshape(1, 32), W2r)
  agg2 = _seg_sum(_pad_rows(p2), src3, dst3, 32, False)[0]
  h2 = _t2(agg2, cnt, r2)
  agg3 = _seg_sum(_pad_rows(h2), src3, dst3, 32, False)[0]
  return _t3(agg3, cnt, h2, W3l, b3.reshape(1, 64), W3r)


# speedup vs baseline: 5.4532x; 5.4532x over previous
import functools

import jax
import jax.numpy as jnp
from jax import lax
from jax.experimental import pallas as pl
from jax.experimental.pallas import tpu as pltpu
from jax.experimental.pallas import tpu_sc as plsc

_N = 10000
_E = 320000
_NC = 2
_NS = 16
_NW = _NC * _NS
_C = 128
_NCH = -(-_E // (_NW * _C))
_EPAD = _NW * _NCH * _C
_RPT = 640
_NPAD = _RPT * _NS
_PAD_NODE = _N + 16

_mesh = plsc.VectorSubcoreMesh(core_axis_name="c", subcore_axis_name="s")


def _cnt_kernel(dst3):
  def body(dst_h, ones_h, zc_h, cnt_o, didx, ones_v, cacc):
    c = lax.axis_index("c")
    s = lax.axis_index("s")
    wid = c * _NS + s
    r0 = s * _RPT
    pltpu.sync_copy(zc_h.at[pl.ds(r0, _RPT)], cacc.at[pl.ds(r0, _RPT)])
    pltpu.sync_copy(ones_h, ones_v)
    plsc.subcore_barrier()

    def chunk(j, carry):
      pltpu.sync_copy(dst_h.at[wid, j], didx)
      pltpu.sync_copy(ones_v, cacc.at[didx], add=True)
      return carry

    lax.fori_loop(0, _NCH, chunk, 0)
    plsc.subcore_barrier()
    pltpu.sync_copy(cacc.at[pl.ds(r0, _RPT)], cnt_o.at[c, pl.ds(r0, _RPT)])

  f = pl.kernel(
      body,
      out_type=jax.ShapeDtypeStruct((_NC, _NPAD, 8), jnp.float32),
      mesh=_mesh,
      compiler_params=pltpu.CompilerParams(use_tc_tiling_on_sc=False),
      scratch_types=(
          pltpu.VMEM((_C,), jnp.int32),
          pltpu.VMEM((_C, 8), jnp.float32),
          pltpu.VMEM_SHARED((_NPAD, 8), jnp.float32),
      ),
  )
  return f(dst3, jnp.ones((_C, 8), jnp.float32),
           jnp.zeros((_NPAD, 8), jnp.float32))


def _seg_sum(feat, src3, dst3, d):
  def body(feat_h, src_h, dst_h, zf_h, agg_o, sidx, didx, rows, acc, sem):
    c = lax.axis_index("c")
    s = lax.axis_index("s")
    wid = c * _NS + s
    r0 = s * _RPT
    pltpu.sync_copy(zf_h.at[pl.ds(r0, _RPT)], acc.at[pl.ds(r0, _RPT)])
    plsc.subcore_barrier()

    def chunk(j, carry):
      pltpu.sync_copy(src_h.at[wid, j], sidx)
      pltpu.sync_copy(dst_h.at[wid, j], didx)
      pltpu.async_copy(feat_h.at[sidx], rows, sem).wait()
      pltpu.sync_copy(rows, acc.at[didx], add=True)
      return carry

    lax.fori_loop(0, _NCH, chunk, 0)
    plsc.subcore_barrier()
    pltpu.sync_copy(acc.at[pl.ds(r0, _RPT)], agg_o.at[c, pl.ds(r0, _RPT)])

  f = pl.kernel(
      body,
      out_type=jax.ShapeDtypeStruct((_NC, _NPAD, d), jnp.float32),
      mesh=_mesh,
      compiler_params=pltpu.CompilerParams(use_tc_tiling_on_sc=False),
      scratch_types=(
          pltpu.VMEM((_C,), jnp.int32),
          pltpu.VMEM((_C,), jnp.int32),
          pltpu.VMEM((_C, d), jnp.float32),
          pltpu.VMEM_SHARED((_NPAD, d), jnp.float32),
          pltpu.SemaphoreType.DMA,
      ),
  )
  return f(feat, src3, dst3, jnp.zeros((_NPAD, d), jnp.float32))


def _dot_t(a, w):
  return lax.dot_general(a, w, (((1,), (1,)), ((), ())),
                         precision=lax.Precision.HIGHEST,
                         preferred_element_type=jnp.float32)


def _mean(agg_r, cnt_r):
  cnt = cnt_r[0, :, 0:1] + cnt_r[1, :, 0:1]
  inv = 1.0 / jnp.maximum(cnt, 1.0)
  return (agg_r[0] + agg_r[1]) * inv


def _elu(h):
  return jnp.where(h > 0, h, jnp.exp(jnp.minimum(h, 0.0)) - 1.0)


_BM = 1000


def _t1_body(x_r, a_r, c_r, w1l_r, b1_r, w1r_r, w2l_r, b2_r, w2r_r,
             p2_o, r2_o):
  mean = _mean(a_r, c_r)
  h1 = _dot_t(mean, w1l_r[...]) + b1_r[...] + _dot_t(x_r[...], w1r_r[...])
  h1 = _elu(h1)
  p2_o[...] = _dot_t(h1, w2l_r[...])
  r2_o[...] = _dot_t(h1, w2r_r[...]) + b2_r[...]


def _t1(x, agg1, cnt, w1l, b1, w1r, w2l, b2, w2r):
  grid = (_N // _BM,)
  full = lambda shape: pl.BlockSpec(shape, lambda i: (0,) * len(shape))
  return pl.pallas_call(
      _t1_body,
      grid=grid,
      in_specs=[
          pl.BlockSpec((_BM, 128), lambda i: (i, 0)),
          pl.BlockSpec((_NC, _BM, 128), lambda i: (0, i, 0)),
          pl.BlockSpec((_NC, _BM, 8), lambda i: (0, i, 0)),
          full((256, 128)), full((1, 256)), full((256, 128)),
          full((32, 256)), full((1, 32)), full((32, 256)),
      ],
      out_specs=[
          pl.BlockSpec((_BM, 32), lambda i: (i, 0)),
          pl.BlockSpec((_BM, 32), lambda i: (i, 0)),
      ],
      out_shape=[
          jax.ShapeDtypeStruct((_N, 32), jnp.float32),
          jax.ShapeDtypeStruct((_N, 32), jnp.float32),
      ],
  )(x, agg1, cnt, w1l, b1, w1r, w2l, b2, w2r)


def _t2_body(a_r, c_r, r2_r, h2_o):
  h2_o[...] = _elu(_mean(a_r, c_r) + r2_r[...])


def _t2(agg2, cnt, r2):
  grid = (_N // _BM,)
  return pl.pallas_call(
      _t2_body,
      grid=grid,
      in_specs=[
          pl.BlockSpec((_NC, _BM, 32), lambda i: (0, i, 0)),
          pl.BlockSpec((_NC, _BM, 8), lambda i: (0, i, 0)),
          pl.BlockSpec((_BM, 32), lambda i: (i, 0)),
      ],
      out_specs=pl.BlockSpec((_BM, 32), lambda i: (i, 0)),
      out_shape=jax.ShapeDtypeStruct((_N, 32), jnp.float32),
  )(agg2, cnt, r2)


def _t3_body(a_r, c_r, h2_r, w3l_r, b3_r, w3r_r, out_o):
  mean = _mean(a_r, c_r)
  out_o[...] = (_dot_t(mean, w3l_r[...]) + b3_r[...]
                + _dot_t(h2_r[...], w3r_r[...]))


def _t3(agg3, cnt, h2, w3l, b3, w3r):
  grid = (_N // _BM,)
  full = lambda shape: pl.BlockSpec(shape, lambda i: (0,) * len(shape))
  return pl.pallas_call(
      _t3_body,
      grid=grid,
      in_specs=[
          pl.BlockSpec((_NC, _BM, 32), lambda i: (0, i, 0)),
          pl.BlockSpec((_NC, _BM, 8), lambda i: (0, i, 0)),
          pl.BlockSpec((_BM, 32), lambda i: (i, 0)),
          full((64, 32)), full((1, 64)), full((64, 32)),
      ],
      out_specs=pl.BlockSpec((_BM, 64), lambda i: (i, 0)),
      out_shape=jax.ShapeDtypeStruct((_N, 64), jnp.float32),
  )(agg3, cnt, h2, w3l, b3, w3r)


def _pad_rows(a):
  return jnp.pad(a, ((0, _NPAD - _N), (0, 0)))


def kernel(x, edge_index, W1l, b1, W1r, W2l, b2, W2r, W3l, b3, W3r):
  ei = edge_index.astype(jnp.int32)
  ei = jnp.pad(ei, ((0, 0), (0, _EPAD - _E)), constant_values=_PAD_NODE)
  src3 = ei[0].reshape(_NW, _NCH, _C)
  dst3 = ei[1].reshape(_NW, _NCH, _C)

  cnt = _cnt_kernel(dst3)
  agg1 = _seg_sum(_pad_rows(x), src3, dst3, 128)
  p2, r2 = _t1(x, agg1, cnt, W1l, b1.reshape(1, 256), W1r,
               W2l, b2.reshape(1, 32), W2r)
  agg2 = _seg_sum(_pad_rows(p2), src3, dst3, 32)
  h2 = _t2(agg2, cnt, r2)
  agg3 = _seg_sum(_pad_rows(h2), src3, dst3, 32)
  return _t3(agg3, cnt, h2, W3l, b3.reshape(1, 64), W3r)


# speedup vs baseline: 8.5573x; 1.5692x over previous
import functools

import jax
import jax.numpy as jnp
from jax import lax
from jax.experimental import pallas as pl
from jax.experimental.pallas import tpu as pltpu
from jax.experimental.pallas import tpu_sc as plsc

_N = 10000
_E = 320000
_NC = 2
_NS = 16
_NW = _NC * _NS
_C = 128
_NCH = -(-_E // (_NW * _C))
_EPAD = _NW * _NCH * _C
_RPT = 640
_NPAD = _RPT * _NS
_PAD_NODE = _N + 16

_mesh = plsc.VectorSubcoreMesh(core_axis_name="c", subcore_axis_name="s")


def _seg_sum(feat, idx3, d, with_cnt):
  def body(*refs):
    if with_cnt:
      (feat_h, idx_h, zf_h, ones_h, zc_h, agg_o, cnt_o,
       i0, i1, i2, i3, r0b, r1b, acc, ones_v, cacc,
       is0, is1, is2, is3, gs0, gs1, ss0, ss1, cs0, cs1) = refs
      csem = [cs0, cs1]
    else:
      (feat_h, idx_h, zf_h, agg_o,
       i0, i1, i2, i3, r0b, r1b, acc,
       is0, is1, is2, is3, gs0, gs1, ss0, ss1) = refs
    idx2 = [i0, i1, i2, i3]
    rows = [r0b, r1b]
    isem = [is0, is1, is2, is3]
    gsem = [gs0, gs1]
    ssem = [ss0, ss1]

    c = lax.axis_index("c")
    s = lax.axis_index("s")
    wid = c * _NS + s
    r0 = s * _RPT

    def idx_load(j, q):
      return pltpu.async_copy(idx_h.at[wid, j], idx2[q], isem[q])

    def gather(j, q, b):
      return pltpu.async_copy(feat_h.at[idx2[q].at[0]], rows[b], gsem[b])

    def scatter(b, q):
      pltpu.async_copy(rows[b], acc.at[idx2[q].at[1]], ssem[b], add=True)
      if with_cnt:
        pltpu.async_copy(ones_v, cacc.at[idx2[q].at[1]], csem[b], add=True)

    def wait_scatter(b, q):
      pltpu.make_async_copy(rows[b], acc.at[idx2[q].at[1]], ssem[b]).wait()
      if with_cnt:
        pltpu.make_async_copy(ones_v, cacc.at[idx2[q].at[1]], csem[b]).wait()

    pltpu.sync_copy(zf_h.at[pl.ds(r0, _RPT)], acc.at[pl.ds(r0, _RPT)])
    if with_cnt:
      pltpu.sync_copy(zc_h.at[pl.ds(r0, _RPT)], cacc.at[pl.ds(r0, _RPT)])
      pltpu.sync_copy(ones_h, ones_v)
    d0 = idx_load(0, 0)
    idx_load(1, 1)
    idx_load(2, 2)
    plsc.subcore_barrier()
    d0.wait()
    gather(0, 0, 0)

    def iter4(jj, carry):
      for q in range(4):
        j = jj * 4 + q
        b = q % 2

        @pl.when(j < _NCH)
        def _():
          pltpu.make_async_copy(feat_h.at[idx2[q].at[0]], rows[b],
                                gsem[b]).wait()
          scatter(b, q)

          @pl.when(j >= 1)
          def _():
            wait_scatter(1 - b, (q + 3) % 4)

          @pl.when(j + 1 < _NCH)
          def _():
            pltpu.make_async_copy(idx_h.at[wid, 0], idx2[(q + 1) % 4],
                                  isem[(q + 1) % 4]).wait()
            gather(j + 1, (q + 1) % 4, 1 - b)

          @pl.when(j + 3 < _NCH)
          def _():
            idx_load(j + 3, (q + 3) % 4)
      return carry

    lax.fori_loop(0, (_NCH + 3) // 4, iter4, 0)
    wait_scatter((_NCH - 1) % 2, (_NCH - 1) % 4)
    plsc.subcore_barrier()
    pltpu.sync_copy(acc.at[pl.ds(r0, _RPT)], agg_o.at[c, pl.ds(r0, _RPT)])
    if with_cnt:
      pltpu.sync_copy(cacc.at[pl.ds(r0, _RPT)], cnt_o.at[c, pl.ds(r0, _RPT)])

  outs = [jax.ShapeDtypeStruct((_NC, _NPAD, d), jnp.float32)]
  inputs = [feat, idx3, jnp.zeros((_NPAD, d), jnp.float32)]
  scratch = (
      [pltpu.VMEM((2, _C), jnp.int32) for _ in range(4)]
      + [pltpu.VMEM((_C, d), jnp.float32) for _ in range(2)]
      + [pltpu.VMEM_SHARED((_NPAD, d), jnp.float32)]
  )
  if with_cnt:
    outs.append(jax.ShapeDtypeStruct((_NC, _NPAD, 8), jnp.float32))
    inputs += [jnp.ones((_C, 8), jnp.float32),
               jnp.zeros((_NPAD, 8), jnp.float32)]
    scratch += [pltpu.VMEM((_C, 8), jnp.float32),
                pltpu.VMEM_SHARED((_NPAD, 8), jnp.float32)]
  nsem = 10 if with_cnt else 8
  scratch += [pltpu.SemaphoreType.DMA for _ in range(nsem)]

  f = pl.kernel(
      body,
      out_type=tuple(outs),
      mesh=_mesh,
      compiler_params=pltpu.CompilerParams(use_tc_tiling_on_sc=False),
      scratch_types=tuple(scratch),
  )
  return f(*inputs)


def _dot_t(a, w):
  return lax.dot_general(a, w, (((1,), (1,)), ((), ())),
                         precision=lax.Precision.HIGHEST,
                         preferred_element_type=jnp.float32)


def _mean(agg_r, cnt_r):
  cnt = cnt_r[0, :, 0:1] + cnt_r[1, :, 0:1]
  inv = 1.0 / jnp.maximum(cnt, 1.0)
  return (agg_r[0] + agg_r[1]) * inv


def _elu(h):
  return jnp.where(h > 0, h, jnp.exp(jnp.minimum(h, 0.0)) - 1.0)


_BM = 1000


def _t1_body(x_r, a_r, c_r, w1l_r, b1_r, w1r_r, w2l_r, b2_r, w2r_r,
             p2_o, r2_o):
  mean = _mean(a_r, c_r)
  h1 = _dot_t(mean, w1l_r[...]) + b1_r[...] + _dot_t(x_r[...], w1r_r[...])
  h1 = _elu(h1)
  p2_o[...] = _dot_t(h1, w2l_r[...])
  r2_o[...] = _dot_t(h1, w2r_r[...]) + b2_r[...]


def _t1(x, agg1, cnt, w1l, b1, w1r, w2l, b2, w2r):
  grid = (_N // _BM,)
  full = lambda shape: pl.BlockSpec(shape, lambda i: (0,) * len(shape))
  return pl.pallas_call(
      _t1_body,
      grid=grid,
      in_specs=[
          pl.BlockSpec((_BM, 128), lambda i: (i, 0)),
          pl.BlockSpec((_NC, _BM, 128), lambda i: (0, i, 0)),
          pl.BlockSpec((_NC, _BM, 8), lambda i: (0, i, 0)),
          full((256, 128)), full((1, 256)), full((256, 128)),
          full((32, 256)), full((1, 32)), full((32, 256)),
      ],
      out_specs=[
          pl.BlockSpec((_BM, 32), lambda i: (i, 0)),
          pl.BlockSpec((_BM, 32), lambda i: (i, 0)),
      ],
      out_shape=[
          jax.ShapeDtypeStruct((_N, 32), jnp.float32),
          jax.ShapeDtypeStruct((_N, 32), jnp.float32),
      ],
  )(x, agg1, cnt, w1l, b1, w1r, w2l, b2, w2r)


def _t2_body(a_r, c_r, r2_r, h2_o):
  h2_o[...] = _elu(_mean(a_r, c_r) + r2_r[...])


def _t2(agg2, cnt, r2):
  grid = (_N // _BM,)
  return pl.pallas_call(
      _t2_body,
      grid=grid,
      in_specs=[
          pl.BlockSpec((_NC, _BM, 32), lambda i: (0, i, 0)),
          pl.BlockSpec((_NC, _BM, 8), lambda i: (0, i, 0)),
          pl.BlockSpec((_BM, 32), lambda i: (i, 0)),
      ],
      out_specs=pl.BlockSpec((_BM, 32), lambda i: (i, 0)),
      out_shape=jax.ShapeDtypeStruct((_N, 32), jnp.float32),
  )(agg2, cnt, r2)


def _t3_body(a_r, c_r, h2_r, w3l_r, b3_r, w3r_r, out_o):
  mean = _mean(a_r, c_r)
  out_o[...] = (_dot_t(mean, w3l_r[...]) + b3_r[...]
                + _dot_t(h2_r[...], w3r_r[...]))


def _t3(agg3, cnt, h2, w3l, b3, w3r):
  grid = (_N // _BM,)
  full = lambda shape: pl.BlockSpec(shape, lambda i: (0,) * len(shape))
  return pl.pallas_call(
      _t3_body,
      grid=grid,
      in_specs=[
          pl.BlockSpec((_NC, _BM, 32), lambda i: (0, i, 0)),
          pl.BlockSpec((_NC, _BM, 8), lambda i: (0, i, 0)),
          pl.BlockSpec((_BM, 32), lambda i: (i, 0)),
          full((64, 32)), full((1, 64)), full((64, 32)),
      ],
      out_specs=pl.BlockSpec((_BM, 64), lambda i: (i, 0)),
      out_shape=jax.ShapeDtypeStruct((_N, 64), jnp.float32),
  )(agg3, cnt, h2, w3l, b3, w3r)


def _pad_rows(a):
  return jnp.pad(a, ((0, _NPAD - _N), (0, 0)))


def kernel(x, edge_index, W1l, b1, W1r, W2l, b2, W2r, W3l, b3, W3r):
  ei = edge_index.astype(jnp.int32)
  ei = jnp.pad(ei, ((0, 0), (0, _EPAD - _E)), constant_values=_PAD_NODE)
  idx3 = jnp.stack([ei[0].reshape(_NW, _NCH, _C),
                    ei[1].reshape(_NW, _NCH, _C)], axis=2)

  agg1, cnt = _seg_sum(_pad_rows(x), idx3, 128, True)
  p2, r2 = _t1(x, agg1, cnt, W1l, b1.reshape(1, 256), W1r,
               W2l, b2.reshape(1, 32), W2r)
  agg2 = _seg_sum(_pad_rows(p2), idx3, 32, False)[0]
  h2 = _t2(agg2, cnt, r2)
  agg3 = _seg_sum(_pad_rows(h2), idx3, 32, False)[0]
  return _t3(agg3, cnt, h2, W3l, b3.reshape(1, 64), W3r)
